# baseline (speedup 1.0000x reference)
import functools

import jax
import jax.numpy as jnp
from jax import lax
from jax.experimental import pallas as pl
from jax.experimental.pallas import tpu as pltpu
from jax.experimental.pallas import tpu_sc as plsc

L = 16


def _tree(op, vals):
    vals = list(vals)
    while len(vals) > 1:
        nxt = [op(vals[i], vals[i + 1]) for i in range(0, len(vals) - 1, 2)]
        if len(vals) % 2:
            nxt.append(vals[-1])
        vals = nxt
    return vals[0]


@functools.lru_cache(maxsize=None)
def _build(B, IN, OUT, K, NOPS):
    mesh = plsc.VectorSubcoreMesh(core_axis_name="c", subcore_axis_name="s")
    NC, NS = mesh.num_cores, mesh.num_subcores
    NW = NC * NS
    R = L
    assert B % (NW * R) == 0 and OUT % L == 0 and L % K == 0
    n_chunks = B // R
    nc_per_w = n_chunks // NW
    n_groups = OUT // L
    FPR = L // K
    n_idx_rows = OUT // FPR

    @functools.partial(
        pl.kernel,
        mesh=mesh,
        compiler_params=pltpu.CompilerParams(
            use_tc_tiling_on_sc=False, needs_layout_passes=False
        ),
        out_type=jax.ShapeDtypeStruct((n_chunks, OUT, R), jnp.float32),
        scratch_types=[
            pltpu.VMEM((n_idx_rows, L), jnp.int32),
            pltpu.VMEM((NOPS, OUT), jnp.float32),
            pltpu.VMEM((OUT,), jnp.int32),
            pltpu.VMEM((OUT,), jnp.int32),
            pltpu.VMEM((IN, R), jnp.float32),
            pltpu.VMEM((OUT, R), jnp.float32),
        ],
    )
    def sc_kernel(xq_hbm, wt_hbm, idx2_hbm, perm_hbm, oq_hbm,
                  idx_v, w_v, id_v, perm_v, xtb, otb):
        wid = lax.axis_index("s") * NC + lax.axis_index("c")
        pltpu.sync_copy(idx2_hbm, idx_v)
        pltpu.sync_copy(wt_hbm, w_v)
        pltpu.sync_copy(perm_hbm, perm_v)

        one = jnp.full((L,), 1.0, jnp.float32)

        def opid_body(g, _):
            s = pl.ds(g * L, L)
            w0, w1, w2, w3 = w_v[0, s], w_v[1, s], w_v[2, s], w_v[3, s]
            i0 = jnp.full((L,), 0, jnp.int32)
            b1 = w1 > w0
            m01 = jnp.maximum(w0, w1)
            i01 = jnp.where(b1, jnp.full((L,), 1, jnp.int32), i0)
            b2 = w2 > m01
            m012 = jnp.maximum(m01, w2)
            i012 = jnp.where(b2, jnp.full((L,), 2, jnp.int32), i01)
            b3 = w3 > m012
            id_v[s] = jnp.where(b3, jnp.full((L,), 3, jnp.int32), i012)
            return _

        lax.fori_loop(0, n_groups, opid_body, 0, unroll=False)

        def chunk_body(c, _):
            g = wid * nc_per_w + c
            pltpu.sync_copy(xq_hbm.at[g], xtb)

            @plsc.parallel_loop(0, n_groups, 1)
            def group_body(gi):
                s = pl.ds(gi * L, L)
                cv = perm_v[s]
                opid = id_v[s]
                sel_min = jnp.min(opid)
                sel_max = jnp.max(opid)
                sel = jnp.where(sel_min == sel_max, sel_min,
                                jnp.int32(NOPS))

                def feature_vals(iv, h):
                    return [xtb[iv[h * K + j]] for j in range(K)]

                def uniform_arm(redop, pre, post):
                    def arm():
                        for p in range(L // FPR):
                            iv = idx_v[gi * (L // FPR) + p]
                            for h in range(FPR):
                                gv = feature_vals(iv, h)
                                res = post(_tree(redop, [pre(v) for v in gv]))
                                otb[cv[p * FPR + h]] = res
                    return arm

                def mixed_arm():
                    for p in range(L // FPR):
                        iv = idx_v[gi * (L // FPR) + p]
                        for h in range(FPR):
                            f = p * FPR + h
                            gv = feature_vals(iv, h)
                            opf = opid[f]
                            mn = _tree(jnp.minimum, gv)
                            mx = _tree(jnp.maximum, gv)
                            pr = _tree(lax.mul, gv)
                            q = _tree(lax.mul, [one - v for v in gv])
                            r01 = jnp.where(opf == 1, mx, mn)
                            r23 = jnp.where(opf == 3, one - q, pr)
                            otb[cv[f]] = jnp.where(opf >= 2, r23, r01)

                ident = lambda v: v
                lax.switch(sel, [
                    uniform_arm(jnp.minimum, ident, ident),
                    uniform_arm(jnp.maximum, ident, ident),
                    uniform_arm(lax.mul, ident, ident),
                    uniform_arm(lax.mul, lambda v: one - v, lambda v: one - v),
                    mixed_arm,
                ])
            pltpu.sync_copy(otb, oq_hbm.at[g])
            return _

        lax.fori_loop(0, nc_per_w, chunk_body, 0, unroll=False)

    return sc_kernel


def kernel(x, weights, connection_indices):
    B, IN = x.shape
    OUT, NOPS = weights.shape
    K = connection_indices.shape[1]
    sc_kernel = _build(B, IN, OUT, K, NOPS)
    opid = jnp.argmax(weights, axis=-1)
    perm = jnp.argsort(opid, stable=True).astype(jnp.int32)
    wt = jnp.transpose(weights[perm])
    idx2 = connection_indices[perm].reshape(OUT * K // 16, 16)
    xq = jnp.swapaxes(x.reshape(B // 16, 16, IN), 1, 2)
    oq = sc_kernel(xq, wt, idx2, perm)
    return jnp.swapaxes(oq, 1, 2).reshape(B, OUT)


# speedup vs baseline: 1.2412x; 1.2412x over previous
import functools

import jax
import jax.numpy as jnp
from jax import lax
from jax.experimental import pallas as pl
from jax.experimental.pallas import tpu as pltpu
from jax.experimental.pallas import tpu_sc as plsc

L = 16


def _tree(op, vals):
    vals = list(vals)
    while len(vals) > 1:
        nxt = [op(vals[i], vals[i + 1]) for i in range(0, len(vals) - 1, 2)]
        if len(vals) % 2:
            nxt.append(vals[-1])
        vals = nxt
    return vals[0]


@functools.lru_cache(maxsize=None)
def _build(B, IN, OUT, K, NOPS):
    mesh = plsc.VectorSubcoreMesh(core_axis_name="c", subcore_axis_name="s")
    NC, NS = mesh.num_cores, mesh.num_subcores
    NW = NC * NS
    R = L
    assert B % (NW * R) == 0 and OUT % (8 * L) == 0 and L % K == 0
    n_chunks = B // R
    nc_per_w = n_chunks // NW
    n_groups = OUT // L
    FPR = L // K
    n_idx_rows = OUT // FPR
    SCAT = 128

    @functools.partial(
        pl.kernel,
        mesh=mesh,
        compiler_params=pltpu.CompilerParams(
            use_tc_tiling_on_sc=False, needs_layout_passes=False
        ),
        out_type=jax.ShapeDtypeStruct((n_chunks * OUT, R), jnp.float32),
        scratch_types=[
            pltpu.VMEM((n_idx_rows, L), jnp.int32),
            pltpu.VMEM((NOPS, OUT), jnp.float32),
            pltpu.VMEM((OUT,), jnp.int32),
            pltpu.VMEM((OUT,), jnp.int32),
            pltpu.VMEM((IN * R,), jnp.float32),
            pltpu.VMEM((OUT, R), jnp.float32),
            pltpu.VMEM((OUT // SCAT, SCAT), jnp.int32),
            pltpu.SemaphoreType.DMA,
        ],
    )
    def sc_kernel(xq_hbm, wt_hbm, idx16_hbm, perm_hbm, oqf_hbm,
                  idx_v, w_v, id_v, perm_v, xtb, otb, sidx, sem):
        wid = lax.axis_index("s") * NC + lax.axis_index("c")
        pltpu.sync_copy(idx16_hbm, idx_v)
        pltpu.sync_copy(wt_hbm, w_v)
        pltpu.sync_copy(perm_hbm, perm_v)

        one = jnp.full((L,), 1.0, jnp.float32)
        iota = lax.iota(jnp.int32, L)
        lanes = [jnp.full((L,), n, jnp.int32) for n in range(L)]

        def opid_body(g, _):
            s = pl.ds(g * L, L)
            w0, w1, w2, w3 = w_v[0, s], w_v[1, s], w_v[2, s], w_v[3, s]
            i0 = jnp.full((L,), 0, jnp.int32)
            b1 = w1 > w0
            m01 = jnp.maximum(w0, w1)
            i01 = jnp.where(b1, jnp.full((L,), 1, jnp.int32), i0)
            b2 = w2 > m01
            m012 = jnp.maximum(m01, w2)
            i012 = jnp.where(b2, jnp.full((L,), 2, jnp.int32), i01)
            b3 = w3 > m012
            id_v[s] = jnp.where(b3, jnp.full((L,), 3, jnp.int32), i012)
            return _

        lax.fori_loop(0, n_groups, opid_body, 0, unroll=False)

        gdn = lax.GatherDimensionNumbers(
            offset_dims=(), collapsed_slice_dims=(0,), start_index_map=(0,))

        def lane_splat(vec, n):
            return lax.gather(
                vec, lanes[n][:, None], gdn, (1,),
                mode=lax.GatherScatterMode.PROMISE_IN_BOUNDS)

        def chunk_body(c, _):
            g = wid * nc_per_w + c
            pltpu.sync_copy(xq_hbm.at[g], xtb)

            goff = jnp.broadcast_to(g * OUT, (L,)).astype(jnp.int32)
            for q in range(OUT // SCAT):
                for m in range(SCAT // L):
                    sidx[q, pl.ds(m * L, L)] = (
                        perm_v[pl.ds(q * SCAT + m * L, L)] + goff)

            @plsc.parallel_loop(0, n_groups, 1)
            def group_body(gi):
                s = pl.ds(gi * L, L)
                opid = id_v[s]
                sel_min = jnp.min(opid)
                sel_max = jnp.max(opid)
                sel = jnp.where(sel_min == sel_max, sel_min,
                                jnp.int32(NOPS))

                def feature_vals(iv, h):
                    return [
                        plsc.load_gather(
                            xtb, [lane_splat(iv, h * K + j) + iota])
                        for j in range(K)
                    ]

                def uniform_arm(redop, pre, post):
                    def arm():
                        for p in range(L // FPR):
                            iv = idx_v[gi * (L // FPR) + p]
                            for h in range(FPR):
                                gv = feature_vals(iv, h)
                                res = post(_tree(redop, [pre(v) for v in gv]))
                                otb[gi * L + p * FPR + h] = res
                    return arm

                def mixed_arm():
                    for p in range(L // FPR):
                        iv = idx_v[gi * (L // FPR) + p]
                        for h in range(FPR):
                            f = p * FPR + h
                            gv = feature_vals(iv, h)
                            ops = lane_splat(opid, f)
                            mn = _tree(jnp.minimum, gv)
                            mx = _tree(jnp.maximum, gv)
                            pr = _tree(lax.mul, gv)
                            q = _tree(lax.mul, [one - v for v in gv])
                            r01 = jnp.where(ops == 1, mx, mn)
                            r23 = jnp.where(ops == 3, one - q, pr)
                            otb[gi * L + f] = jnp.where(ops >= 2, r23, r01)

                ident = lambda v: v
                lax.switch(sel, [
                    uniform_arm(jnp.minimum, ident, ident),
                    uniform_arm(jnp.maximum, ident, ident),
                    uniform_arm(lax.mul, ident, ident),
                    uniform_arm(lax.mul, lambda v: one - v, lambda v: one - v),
                    mixed_arm,
                ])

            copies = [
                pltpu.async_copy(
                    otb.at[pl.ds(q * SCAT, SCAT)], oqf_hbm.at[sidx.at[q]], sem)
                for q in range(OUT // SCAT)
            ]
            for cp in copies:
                cp.wait()
            return _

        lax.fori_loop(0, nc_per_w, chunk_body, 0, unroll=False)

    return sc_kernel


def kernel(x, weights, connection_indices):
    B, IN = x.shape
    OUT, NOPS = weights.shape
    K = connection_indices.shape[1]
    sc_kernel = _build(B, IN, OUT, K, NOPS)
    opid = jnp.argmax(weights, axis=-1)
    perm = jnp.argsort(opid, stable=True).astype(jnp.int32)
    wt = jnp.transpose(weights[perm])
    idx16 = (connection_indices[perm] * 16).reshape(OUT * K // 16, 16)
    xq = jnp.swapaxes(x.reshape(B // 16, 16, IN), 1, 2).reshape(B // 16, IN * 16)
    oqf = sc_kernel(xq, wt, idx16, perm)
    oq = oqf.reshape(B // 16, OUT, 16)
    return jnp.swapaxes(oq, 1, 2).reshape(B, OUT)


# speedup vs baseline: 1.4746x; 1.1880x over previous
import functools

import jax
import jax.numpy as jnp
from jax import lax
from jax.experimental import pallas as pl
from jax.experimental.pallas import tpu as pltpu
from jax.experimental.pallas import tpu_sc as plsc

L = 16


def _tree(op, vals):
    vals = list(vals)
    while len(vals) > 1:
        nxt = [op(vals[i], vals[i + 1]) for i in range(0, len(vals) - 1, 2)]
        if len(vals) % 2:
            nxt.append(vals[-1])
        vals = nxt
    return vals[0]


@functools.lru_cache(maxsize=None)
def _build(B, IN, OUT, K, NOPS):
    mesh = plsc.VectorSubcoreMesh(core_axis_name="c", subcore_axis_name="s")
    NC, NS = mesh.num_cores, mesh.num_subcores
    NW = NC * NS
    R = L
    assert B % (NW * R) == 0 and OUT % (8 * L) == 0 and L % K == 0
    n_chunks = B // R
    nc_per_w = n_chunks // NW
    n_groups = OUT // L
    FPR = L // K
    n_idx_rows = OUT // FPR

    @functools.partial(
        pl.kernel,
        mesh=mesh,
        compiler_params=pltpu.CompilerParams(
            use_tc_tiling_on_sc=False, needs_layout_passes=False
        ),
        out_type=jax.ShapeDtypeStruct((n_chunks, OUT * R), jnp.float32),
        scratch_types=[
            pltpu.VMEM((n_idx_rows, L), jnp.int32),
            pltpu.VMEM((NOPS, OUT), jnp.float32),
            pltpu.VMEM((OUT,), jnp.int32),
            pltpu.VMEM((OUT,), jnp.int32),
            pltpu.VMEM((IN * R,), jnp.float32),
            pltpu.VMEM((OUT * R,), jnp.float32),
        ],
    )
    def sc_kernel(xq_hbm, wt_hbm, idx16_hbm, perm16_hbm, oq_hbm,
                  idx_v, w_v, id_v, perm_v, xtb, otb):
        wid = lax.axis_index("s") * NC + lax.axis_index("c")
        pltpu.sync_copy(idx16_hbm, idx_v)
        pltpu.sync_copy(wt_hbm, w_v)
        pltpu.sync_copy(perm16_hbm, perm_v)

        one = jnp.full((L,), 1.0, jnp.float32)
        iota = lax.iota(jnp.int32, L)
        lanes = [jnp.full((L,), n, jnp.int32) for n in range(L)]

        def opid_body(g, _):
            s = pl.ds(g * L, L)
            w0, w1, w2, w3 = w_v[0, s], w_v[1, s], w_v[2, s], w_v[3, s]
            i0 = jnp.full((L,), 0, jnp.int32)
            b1 = w1 > w0
            m01 = jnp.maximum(w0, w1)
            i01 = jnp.where(b1, jnp.full((L,), 1, jnp.int32), i0)
            b2 = w2 > m01
            m012 = jnp.maximum(m01, w2)
            i012 = jnp.where(b2, jnp.full((L,), 2, jnp.int32), i01)
            b3 = w3 > m012
            id_v[s] = jnp.where(b3, jnp.full((L,), 3, jnp.int32), i012)
            return _

        lax.fori_loop(0, n_groups, opid_body, 0, unroll=False)

        gdn = lax.GatherDimensionNumbers(
            offset_dims=(), collapsed_slice_dims=(0,), start_index_map=(0,))

        def lane_splat(vec, n):
            return lax.gather(
                vec, lanes[n][:, None], gdn, (1,),
                mode=lax.GatherScatterMode.PROMISE_IN_BOUNDS)

        def chunk_body(c, _):
            g = wid * nc_per_w + c
            pltpu.sync_copy(xq_hbm.at[g], xtb)

            @plsc.parallel_loop(0, n_groups, 1)
            def group_body(gi):
                s = pl.ds(gi * L, L)
                opid = id_v[s]
                cv16 = perm_v[s]
                sel_min = jnp.min(opid)
                sel_max = jnp.max(opid)
                sel = jnp.where(sel_min == sel_max, sel_min,
                                jnp.int32(NOPS))

                def feature_vals(iv, h):
                    return [
                        plsc.load_gather(
                            xtb, [lane_splat(iv, h * K + j) + iota])
                        for j in range(K)
                    ]

                def store_res(f, res):
                    tgt = lane_splat(cv16, f) + iota
                    plsc.store_scatter(otb, [tgt], res)

                def uniform_arm(redop, pre, post):
                    def arm():
                        for p in range(L // FPR):
                            iv = idx_v[gi * (L // FPR) + p]
                            for h in range(FPR):
                                gv = feature_vals(iv, h)
                                res = post(_tree(redop, [pre(v) for v in gv]))
                                store_res(p * FPR + h, res)
                    return arm

                def mixed_arm():
                    for p in range(L // FPR):
                        iv = idx_v[gi * (L // FPR) + p]
                        for h in range(FPR):
                            f = p * FPR + h
                            gv = feature_vals(iv, h)
                            ops = lane_splat(opid, f)
                            mn = _tree(jnp.minimum, gv)
                            mx = _tree(jnp.maximum, gv)
                            pr = _tree(lax.mul, gv)
                            q = _tree(lax.mul, [one - v for v in gv])
                            r01 = jnp.where(ops == 1, mx, mn)
                            r23 = jnp.where(ops == 3, one - q, pr)
                            store_res(f, jnp.where(ops >= 2, r23, r01))

                ident = lambda v: v
                lax.switch(sel, [
                    uniform_arm(jnp.minimum, ident, ident),
                    uniform_arm(jnp.maximum, ident, ident),
                    uniform_arm(lax.mul, ident, ident),
                    uniform_arm(lax.mul, lambda v: one - v, lambda v: one - v),
                    mixed_arm,
                ])

            pltpu.sync_copy(otb, oq_hbm.at[g])
            return _

        lax.fori_loop(0, nc_per_w, chunk_body, 0, unroll=False)

    return sc_kernel


def kernel(x, weights, connection_indices):
    B, IN = x.shape
    OUT, NOPS = weights.shape
    K = connection_indices.shape[1]
    sc_kernel = _build(B, IN, OUT, K, NOPS)
    opid = jnp.argmax(weights, axis=-1)
    perm = jnp.argsort(opid, stable=True).astype(jnp.int32)
    wt = jnp.transpose(weights[perm])
    idx16 = (connection_indices[perm] * 16).reshape(OUT * K // 16, 16)
    perm16 = perm * 16
    xq = jnp.swapaxes(x.reshape(B // 16, 16, IN), 1, 2).reshape(B // 16, IN * 16)
    oqf = sc_kernel(xq, wt, idx16, perm16)
    oq = oqf.reshape(B // 16, OUT, 16)
    return jnp.swapaxes(oq, 1, 2).reshape(B, OUT)


# speedup vs baseline: 1.6934x; 1.1484x over previous
import functools

import jax
import jax.numpy as jnp
from jax import lax
from jax.experimental import pallas as pl
from jax.experimental.pallas import tpu as pltpu
from jax.experimental.pallas import tpu_sc as plsc

L = 16


def _tree(op, vals):
    vals = list(vals)
    while len(vals) > 1:
        nxt = [op(vals[i], vals[i + 1]) for i in range(0, len(vals) - 1, 2)]
        if len(vals) % 2:
            nxt.append(vals[-1])
        vals = nxt
    return vals[0]


@functools.lru_cache(maxsize=None)
def _build(B, IN, OUT, K, NOPS):
    mesh = plsc.VectorSubcoreMesh(core_axis_name="c", subcore_axis_name="s")
    NC, NS = mesh.num_cores, mesh.num_subcores
    NW = NC * NS
    assert B % NW == 0
    rows_per_w = B // NW
    R = 16 if rows_per_w % 16 == 0 else rows_per_w
    n_chunks = rows_per_w // R
    n_groups = OUT // L

    @functools.partial(
        pl.kernel,
        mesh=mesh,
        compiler_params=pltpu.CompilerParams(
            use_tc_tiling_on_sc=False, needs_layout_passes=False
        ),
        out_type=jax.ShapeDtypeStruct((B, OUT), jnp.float32),
        scratch_types=[
            pltpu.VMEM((K, OUT), jnp.int32),
            pltpu.VMEM((NOPS, OUT), jnp.float32),
            pltpu.VMEM((OUT,), jnp.int32),
            pltpu.VMEM((OUT,), jnp.int32),
            pltpu.VMEM((R, IN), jnp.float32),
            pltpu.VMEM((R, OUT), jnp.float32),
        ],
    )
    def sc_kernel(x_hbm, wt_hbm, idxt_hbm, perm_hbm, out_hbm,
                  idx_v, w_v, id_v, perm_v, xbuf, obuf):
        wid = lax.axis_index("s") * NC + lax.axis_index("c")
        pltpu.sync_copy(idxt_hbm, idx_v)
        pltpu.sync_copy(wt_hbm, w_v)
        pltpu.sync_copy(perm_hbm, perm_v)

        one = jnp.full((L,), 1.0, jnp.float32)
        rows = [jnp.full((L,), r, jnp.int32) for r in range(R)]

        def opid_body(g, _):
            s = pl.ds(g * L, L)
            w0, w1, w2, w3 = w_v[0, s], w_v[1, s], w_v[2, s], w_v[3, s]
            i0 = jnp.full((L,), 0, jnp.int32)
            b1 = w1 > w0
            m01 = jnp.maximum(w0, w1)
            i01 = jnp.where(b1, jnp.full((L,), 1, jnp.int32), i0)
            b2 = w2 > m01
            m012 = jnp.maximum(m01, w2)
            i012 = jnp.where(b2, jnp.full((L,), 2, jnp.int32), i01)
            b3 = w3 > m012
            id_v[s] = jnp.where(b3, jnp.full((L,), 3, jnp.int32), i012)
            return _

        lax.fori_loop(0, n_groups, opid_body, 0, unroll=False)

        for c in range(n_chunks):
            base = wid * rows_per_w + c * R
            pltpu.sync_copy(x_hbm.at[pl.ds(base, R)], xbuf)

            @plsc.parallel_loop(0, n_groups, 1)
            def group_body(g):
                s = pl.ds(g * L, L)
                idx = [idx_v[k, s] for k in range(K)]
                cols = perm_v[s]
                opid = id_v[s]
                sel_min = jnp.min(opid)
                sel_max = jnp.max(opid)
                sel = jnp.where(sel_min == sel_max, sel_min,
                                jnp.int32(NOPS))

                def uniform_arm(redop, post):
                    def arm():
                        for r in range(R):
                            gv = [plsc.load_gather(xbuf.at[r], [ik])
                                  for ik in idx]
                            plsc.store_scatter(
                                obuf, [rows[r], cols], post(_tree(redop, gv)))
                    return arm

                def coein_arm():
                    for r in range(R):
                        gv = [plsc.load_gather(xbuf.at[r], [ik])
                              for ik in idx]
                        q = _tree(lax.mul, [one - v for v in gv])
                        plsc.store_scatter(obuf, [rows[r], cols], one - q)

                def mixed_arm():
                    is_mx = opid == 1
                    is_co = opid == 3
                    is_pc = opid >= 2
                    for r in range(R):
                        gv = [plsc.load_gather(xbuf.at[r], [ik])
                              for ik in idx]
                        mn = _tree(jnp.minimum, gv)
                        mx = _tree(jnp.maximum, gv)
                        pr = _tree(lax.mul, gv)
                        q = _tree(lax.mul, [one - v for v in gv])
                        r01 = jnp.where(is_mx, mx, mn)
                        r23 = jnp.where(is_co, one - q, pr)
                        plsc.store_scatter(
                            obuf, [rows[r], cols], jnp.where(is_pc, r23, r01))

                lax.switch(sel, [
                    uniform_arm(jnp.minimum, lambda v: v),
                    uniform_arm(jnp.maximum, lambda v: v),
                    uniform_arm(lax.mul, lambda v: v),
                    coein_arm,
                    mixed_arm,
                ])
            pltpu.sync_copy(obuf, out_hbm.at[pl.ds(base, R)])

    return sc_kernel


def kernel(x, weights, connection_indices):
    B, IN = x.shape
    OUT, NOPS = weights.shape
    K = connection_indices.shape[1]
    sc_kernel = _build(B, IN, OUT, K, NOPS)
    opid = jnp.argmax(weights, axis=-1)
    perm = jnp.argsort(opid, stable=True).astype(jnp.int32)
    wt = jnp.transpose(weights[perm])
    idxp = connection_indices[perm]
    order = jnp.argsort(idxp % L, axis=1)
    idxs = jnp.take_along_axis(idxp, order, axis=1)
    lane = (jnp.arange(OUT, dtype=jnp.int32) % L)[:, None]
    pos = (jnp.arange(K, dtype=jnp.int32)[None, :] + lane) % K
    idxt = jnp.transpose(jnp.take_along_axis(idxs, pos, axis=1))
    return sc_kernel(x, wt, idxt, perm)


# speedup vs baseline: 1.8047x; 1.0657x over previous
import functools

import jax
import jax.numpy as jnp
from jax import lax
from jax.experimental import pallas as pl
from jax.experimental.pallas import tpu as pltpu
from jax.experimental.pallas import tpu_sc as plsc

L = 16


def _tree(op, vals):
    vals = list(vals)
    while len(vals) > 1:
        nxt = [op(vals[i], vals[i + 1]) for i in range(0, len(vals) - 1, 2)]
        if len(vals) % 2:
            nxt.append(vals[-1])
        vals = nxt
    return vals[0]


@functools.lru_cache(maxsize=None)
def _build(B, IN, OUT, K, NOPS):
    mesh = plsc.VectorSubcoreMesh(core_axis_name="c", subcore_axis_name="s")
    NC, NS = mesh.num_cores, mesh.num_subcores
    NW = NC * NS
    assert B % NW == 0
    rows_per_w = B // NW
    R = 16 if rows_per_w % 16 == 0 else rows_per_w
    n_chunks = rows_per_w // R
    n_groups = OUT // L

    @functools.partial(
        pl.kernel,
        mesh=mesh,
        compiler_params=pltpu.CompilerParams(
            use_tc_tiling_on_sc=False, needs_layout_passes=False
        ),
        out_type=jax.ShapeDtypeStruct((B, OUT), jnp.float32),
        scratch_types=[
            pltpu.VMEM((K, OUT), jnp.int32),
            pltpu.VMEM((NOPS, OUT), jnp.float32),
            pltpu.VMEM((OUT,), jnp.int32),
            pltpu.VMEM((OUT,), jnp.int32),
            pltpu.VMEM((R, IN), jnp.float32),
            pltpu.VMEM((R, OUT), jnp.float32),
        ],
    )
    def sc_kernel(x_hbm, wt_hbm, idxt_hbm, perm_hbm, out_hbm,
                  idx_v, w_v, id_v, perm_v, xbuf, obuf):
        wid = lax.axis_index("s") * NC + lax.axis_index("c")
        pltpu.sync_copy(idxt_hbm, idx_v)
        pltpu.sync_copy(wt_hbm, w_v)
        pltpu.sync_copy(perm_hbm, perm_v)

        one = jnp.full((L,), 1.0, jnp.float32)
        rows = [jnp.full((L,), r, jnp.int32) for r in range(R)]

        def opid_body(g, _):
            s = pl.ds(g * L, L)
            w0, w1, w2, w3 = w_v[0, s], w_v[1, s], w_v[2, s], w_v[3, s]
            i0 = jnp.full((L,), 0, jnp.int32)
            b1 = w1 > w0
            m01 = jnp.maximum(w0, w1)
            i01 = jnp.where(b1, jnp.full((L,), 1, jnp.int32), i0)
            b2 = w2 > m01
            m012 = jnp.maximum(m01, w2)
            i012 = jnp.where(b2, jnp.full((L,), 2, jnp.int32), i01)
            b3 = w3 > m012
            id_v[s] = jnp.where(b3, jnp.full((L,), 3, jnp.int32), i012)
            return _

        lax.fori_loop(0, n_groups, opid_body, 0, unroll=False)

        for c in range(n_chunks):
            base = wid * rows_per_w + c * R
            pltpu.sync_copy(x_hbm.at[pl.ds(base, R)], xbuf)

            @plsc.parallel_loop(0, n_groups, 1)
            def group_body(g):
                s = pl.ds(g * L, L)
                idx = [idx_v[k, s] for k in range(K)]
                cols = perm_v[s]
                opid = id_v[s]
                sel_min = jnp.min(opid)
                sel_max = jnp.max(opid)
                sel = jnp.where(sel_min == sel_max, sel_min,
                                jnp.int32(NOPS))

                def uniform_arm(redop, post):
                    def arm():
                        for r in range(R):
                            gv = [plsc.load_gather(xbuf.at[r], [ik])
                                  for ik in idx]
                            plsc.store_scatter(
                                obuf, [rows[r], cols], post(_tree(redop, gv)))
                    return arm

                def coein_arm():
                    for r in range(R):
                        gv = [plsc.load_gather(xbuf.at[r], [ik])
                              for ik in idx]
                        q = _tree(lax.mul, [one - v for v in gv])
                        plsc.store_scatter(obuf, [rows[r], cols], one - q)

                def mixed_arm():
                    is_mx = opid == 1
                    is_co = opid == 3
                    is_pc = opid >= 2
                    for r in range(R):
                        gv = [plsc.load_gather(xbuf.at[r], [ik])
                              for ik in idx]
                        mn = _tree(jnp.minimum, gv)
                        mx = _tree(jnp.maximum, gv)
                        pr = _tree(lax.mul, gv)
                        q = _tree(lax.mul, [one - v for v in gv])
                        r01 = jnp.where(is_mx, mx, mn)
                        r23 = jnp.where(is_co, one - q, pr)
                        plsc.store_scatter(
                            obuf, [rows[r], cols], jnp.where(is_pc, r23, r01))

                lax.switch(sel, [
                    uniform_arm(jnp.minimum, lambda v: v),
                    uniform_arm(jnp.maximum, lambda v: v),
                    uniform_arm(lax.mul, lambda v: v),
                    coein_arm,
                    mixed_arm,
                ])
            pltpu.sync_copy(obuf, out_hbm.at[pl.ds(base, R)])

    return sc_kernel


def kernel(x, weights, connection_indices):
    B, IN = x.shape
    OUT, NOPS = weights.shape
    K = connection_indices.shape[1]
    sc_kernel = _build(B, IN, OUT, K, NOPS)
    opid = jnp.argmax(weights, axis=-1)
    perm = jnp.argsort(opid, stable=True).astype(jnp.int32)
    wt = jnp.transpose(weights[perm])
    idxt = jnp.transpose(connection_indices[perm])
    return sc_kernel(x, wt, idxt, perm)


# speedup vs baseline: 1.9066x; 1.0565x over previous
import functools

import jax
import jax.numpy as jnp
from jax import lax
from jax.experimental import pallas as pl
from jax.experimental.pallas import tpu as pltpu
from jax.experimental.pallas import tpu_sc as plsc

L = 16


def _tree(op, vals):
    vals = list(vals)
    while len(vals) > 1:
        nxt = [op(vals[i], vals[i + 1]) for i in range(0, len(vals) - 1, 2)]
        if len(vals) % 2:
            nxt.append(vals[-1])
        vals = nxt
    return vals[0]


@functools.lru_cache(maxsize=None)
def _build(B, IN, OUT, K, NOPS):
    mesh = plsc.VectorSubcoreMesh(core_axis_name="c", subcore_axis_name="s")
    NC, NS = mesh.num_cores, mesh.num_subcores
    NW = NC * NS
    assert B % NW == 0
    rows_per_w = B // NW
    R = 16 if rows_per_w % 16 == 0 else rows_per_w
    n_chunks = rows_per_w // R
    n_groups = OUT // L

    @functools.partial(
        pl.kernel,
        mesh=mesh,
        compiler_params=pltpu.CompilerParams(
            use_tc_tiling_on_sc=False, needs_layout_passes=False
        ),
        out_type=jax.ShapeDtypeStruct((B, OUT), jnp.float32),
        scratch_types=[
            pltpu.VMEM((K, OUT), jnp.int32),
            pltpu.VMEM((OUT * NOPS,), jnp.float32),
            pltpu.VMEM((OUT * K,), jnp.int32),
            pltpu.VMEM((OUT,), jnp.int32),
            pltpu.VMEM((OUT,), jnp.int32),
            pltpu.VMEM((R, IN), jnp.float32),
            pltpu.VMEM((R, OUT), jnp.float32),
        ],
    )
    def sc_kernel(x_hbm, wf_hbm, idxf_hbm, perm_hbm, out_hbm,
                  idx_v, wraw_v, iraw_v, id_v, perm_v, xbuf, obuf):
        wid = lax.axis_index("s") * NC + lax.axis_index("c")
        pltpu.sync_copy(wf_hbm, wraw_v)
        pltpu.sync_copy(idxf_hbm, iraw_v)
        pltpu.sync_copy(perm_hbm, perm_v)

        one = jnp.full((L,), 1.0, jnp.float32)
        rows = [jnp.full((L,), r, jnp.int32) for r in range(R)]

        def opid_body(g, _):
            s = pl.ds(g * L, L)
            p = perm_v[s]
            wbase = p * NOPS
            w0 = plsc.load_gather(wraw_v, [wbase])
            w1 = plsc.load_gather(wraw_v, [wbase + 1])
            w2 = plsc.load_gather(wraw_v, [wbase + 2])
            w3 = plsc.load_gather(wraw_v, [wbase + 3])
            ibase = p * K
            for k in range(K):
                idx_v[k, s] = plsc.load_gather(iraw_v, [ibase + k])
            i0 = jnp.full((L,), 0, jnp.int32)
            b1 = w1 > w0
            m01 = jnp.maximum(w0, w1)
            i01 = jnp.where(b1, jnp.full((L,), 1, jnp.int32), i0)
            b2 = w2 > m01
            m012 = jnp.maximum(m01, w2)
            i012 = jnp.where(b2, jnp.full((L,), 2, jnp.int32), i01)
            b3 = w3 > m012
            id_v[s] = jnp.where(b3, jnp.full((L,), 3, jnp.int32), i012)
            return _

        lax.fori_loop(0, n_groups, opid_body, 0, unroll=False)

        for c in range(n_chunks):
            base = wid * rows_per_w + c * R
            pltpu.sync_copy(x_hbm.at[pl.ds(base, R)], xbuf)

            @plsc.parallel_loop(0, n_groups, 1)
            def group_body(g):
                s = pl.ds(g * L, L)
                idx = [idx_v[k, s] for k in range(K)]
                cols = perm_v[s]
                opid = id_v[s]
                sel_min = jnp.min(opid)
                sel_max = jnp.max(opid)
                sel = jnp.where(sel_min == sel_max, sel_min,
                                jnp.int32(NOPS))

                def uniform_arm(redop, post):
                    def arm():
                        for r in range(R):
                            gv = [plsc.load_gather(xbuf.at[r], [ik])
                                  for ik in idx]
                            plsc.store_scatter(
                                obuf, [rows[r], cols], post(_tree(redop, gv)))
                    return arm

                def coein_arm():
                    for r in range(R):
                        gv = [plsc.load_gather(xbuf.at[r], [ik])
                              for ik in idx]
                        q = _tree(lax.mul, [one - v for v in gv])
                        plsc.store_scatter(obuf, [rows[r], cols], one - q)

                def mixed_arm():
                    is_mx = opid == 1
                    is_co = opid == 3
                    is_pc = opid >= 2
                    for r in range(R):
                        gv = [plsc.load_gather(xbuf.at[r], [ik])
                              for ik in idx]
                        mn = _tree(jnp.minimum, gv)
                        mx = _tree(jnp.maximum, gv)
                        pr = _tree(lax.mul, gv)
                        q = _tree(lax.mul, [one - v for v in gv])
                        r01 = jnp.where(is_mx, mx, mn)
                        r23 = jnp.where(is_co, one - q, pr)
                        plsc.store_scatter(
                            obuf, [rows[r], cols], jnp.where(is_pc, r23, r01))

                lax.switch(sel, [
                    uniform_arm(jnp.minimum, lambda v: v),
                    uniform_arm(jnp.maximum, lambda v: v),
                    uniform_arm(lax.mul, lambda v: v),
                    coein_arm,
                    mixed_arm,
                ])
            pltpu.sync_copy(obuf, out_hbm.at[pl.ds(base, R)])

    return sc_kernel


def kernel(x, weights, connection_indices):
    B, IN = x.shape
    OUT, NOPS = weights.shape
    K = connection_indices.shape[1]
    sc_kernel = _build(B, IN, OUT, K, NOPS)
    opid = jnp.argmax(weights, axis=-1)
    oh = (opid[:, None] == jnp.arange(NOPS, dtype=opid.dtype)[None, :])
    ohi = oh.astype(jnp.int32)
    counts = jnp.sum(ohi, axis=0)
    offsets = jnp.concatenate(
        [jnp.zeros((1,), jnp.int32), jnp.cumsum(counts)[:-1]])
    pos_all = jnp.cumsum(ohi, axis=0) - ohi + offsets[None, :]
    pos = jnp.sum(jnp.where(oh, pos_all, 0), axis=1)
    perm = jnp.zeros((OUT,), jnp.int32).at[pos].set(
        jnp.arange(OUT, dtype=jnp.int32))
    return sc_kernel(x, weights.reshape(-1), connection_indices.reshape(-1),
                     perm)
